# baseline (speedup 1.0000x reference)
import functools

import jax
import jax.numpy as jnp
from jax import lax
from jax.experimental import pallas as pl
from jax.experimental.pallas import tpu as pltpu
from jax.experimental.pallas import tpu_sc as plsc

INPUT_DIM = 1000000
PROJ_DIM = 32
BATCH = 16384

_info = plsc.get_sparse_core_info()
_NC, _NS, _L = _info.num_cores, _info.num_subcores, _info.num_lanes
_NW = _NC * _NS
_BPW = BATCH // _NW


def _sc_body(x0_hbm, x1_hbm, ww_hbm, wc_hbm, out_hbm,
             idx0_v, idx1_v, rows_w, rows_c, out_v, sem0, sem1):
    wid = lax.axis_index("s") * _NC + lax.axis_index("c")
    base = wid * _BPW

    pltpu.sync_copy(x0_hbm.at[pl.ds(base, _BPW)], idx0_v)
    pltpu.sync_copy(x1_hbm.at[pl.ds(base, _BPW)], idx1_v)

    cp0 = pltpu.async_copy(ww_hbm.at[idx0_v], rows_w, sem0)
    cp1 = pltpu.async_copy(wc_hbm.at[idx1_v], rows_c, sem1)
    cp0.wait()
    cp1.wait()

    lane_iota = lax.iota(jnp.int32, _L)

    def block_body(b, _):
        row_idx = lane_iota + b * _L
        acc = jnp.zeros((_L,), jnp.float32)
        for j in range(PROJ_DIM):
            col = jnp.full((_L,), j, jnp.int32)
            gw = plsc.load_gather(rows_w, [row_idx, col])
            gc = plsc.load_gather(rows_c, [row_idx, col])
            acc = acc + gw * gc
        out_v[pl.ds(b * _L, _L)] = 1.0 / (1.0 + jnp.exp(-acc))
        return 0

    lax.fori_loop(0, _BPW // _L, block_body, 0)

    pltpu.sync_copy(out_v, out_hbm.at[pl.ds(base, _BPW)])


@jax.jit
def _run(x0, x1, W_w, W_c):
    mesh = plsc.VectorSubcoreMesh(core_axis_name="c", subcore_axis_name="s")
    f = pl.kernel(
        _sc_body,
        mesh=mesh,
        compiler_params=pltpu.CompilerParams(
            use_tc_tiling_on_sc=False, needs_layout_passes=False),
        out_type=jax.ShapeDtypeStruct((BATCH,), jnp.float32),
        scratch_types=[
            pltpu.VMEM((_BPW,), jnp.int32),
            pltpu.VMEM((_BPW,), jnp.int32),
            pltpu.VMEM((_BPW, PROJ_DIM), jnp.float32),
            pltpu.VMEM((_BPW, PROJ_DIM), jnp.float32),
            pltpu.VMEM((_BPW,), jnp.float32),
            pltpu.SemaphoreType.DMA,
            pltpu.SemaphoreType.DMA,
        ],
    )
    return f(x0, x1, W_w, W_c)


def kernel(X, W_w, W_c):
    x0 = X[:, 0].astype(jnp.int32)
    x1 = X[:, 1].astype(jnp.int32)
    out = _run(x0, x1, W_w, W_c)
    return jnp.reshape(out, (BATCH, 1))
